# baseline (speedup 1.0000x reference)
import jax
import jax.numpy as jnp
from jax.experimental import pallas as pl
from jax.experimental.pallas import tpu as pltpu

HIGH = jax.lax.Precision.HIGHEST


def _fused_kernel(hw, bt, x_ref, wbt_ref, bbt_ref, wfet_ref, bfet_ref,
                  wclt_ref, bclt_ref, emb_ref, met_ref, clu_ref):
    x_bf = x_ref[...].astype(jnp.bfloat16)
    xp = jnp.concatenate([x_bf[k] for k in range(bt)], axis=1)

    r = jnp.dot(wbt_ref[...], xp,
                preferred_element_type=jnp.float32) + bbt_ref[...]
    for k in range(bt):
        emb_ref[k] = r[:, k * hw:(k + 1) * hw]

    n = bt * hw
    col_b = jax.lax.broadcasted_iota(jnp.int32, (n, bt), 0) // hw
    tgt_b = jax.lax.broadcasted_iota(jnp.int32, (n, bt), 1)
    mask_i = jnp.maximum(0, 1 - jnp.abs(col_b - tgt_b))
    mask = mask_i.astype(jnp.float32).astype(jnp.bfloat16)
    xm = jnp.dot(xp, mask, preferred_element_type=jnp.float32) * (1.0 / hw)

    feats = jnp.dot(wfet_ref[...], xm,
                    preferred_element_type=jnp.float32) + bfet_ref[...]
    inv_f = jax.lax.rsqrt(
        jnp.maximum(jnp.sum(feats * feats, axis=0, keepdims=True), 1e-24))
    metric = feats * inv_f

    cluster = jnp.dot(wclt_ref[...], metric,
                      preferred_element_type=jnp.float32) + bclt_ref[...]
    inv_c = jax.lax.rsqrt(
        jnp.maximum(jnp.sum(cluster * cluster, axis=0, keepdims=True), 1e-24))

    met_ref[0] = metric
    clu_ref[0] = cluster * inv_c


def kernel(x_nchw, w_base, b_base, w_feat, b_feat, bn_gamma, bn_beta,
           bn_rm, bn_rv, w_cl, b_cl):
    B, Cin, H, W = x_nchw.shape
    HW = H * W
    Cemb = w_base.shape[1]
    low_dim = w_feat.shape[1]
    n_cluster = w_cl.shape[1]

    bt = B
    for cand in (20, 16, 10, 8, 40, 32, 4, 2):
        if B % cand == 0:
            bt = cand
            break
    n_tiles = B // bt

    x3 = x_nchw.reshape(B, Cin, HW)
    wbt_bf = jnp.transpose(w_base).astype(jnp.bfloat16)
    bbt = b_base.reshape(Cemb, 1)

    w_feat_eff = jnp.dot(w_base, w_feat, precision=HIGH)
    b_feat_eff = jnp.dot(b_base, w_feat, precision=HIGH) + b_feat
    s = bn_gamma * jax.lax.rsqrt(bn_rv + 1e-5)
    w_cl_eff = w_cl * s.reshape(low_dim, 1)
    b_cl_eff = b_cl + jnp.dot(bn_beta - bn_rm * s, w_cl, precision=HIGH)
    wfet = jnp.transpose(w_feat_eff)
    bfet = b_feat_eff.reshape(low_dim, 1)
    wclt = jnp.transpose(w_cl_eff)
    bclt = b_cl_eff.reshape(n_cluster, 1)

    flops = 2 * B * HW * Cin * Cemb + 2 * B * Cin * low_dim \
        + 2 * B * low_dim * n_cluster
    bytes_accessed = 4 * (B * HW * Cin + Cin * low_dim + low_dim
                          + low_dim * n_cluster + n_cluster
                          + B * HW * Cemb + B * (low_dim + n_cluster)) \
        + 2 * Cin * Cemb

    body = lambda *refs: _fused_kernel(HW, bt, *refs)
    emb3, met_t, clu_t = pl.pallas_call(
        body,
        out_shape=(
            jax.ShapeDtypeStruct((B, Cemb, HW), jnp.float32),
            jax.ShapeDtypeStruct((n_tiles, low_dim, bt), jnp.float32),
            jax.ShapeDtypeStruct((n_tiles, n_cluster, bt), jnp.float32),
        ),
        grid=(n_tiles,),
        in_specs=[
            pl.BlockSpec((bt, Cin, HW), lambda i: (i, 0, 0)),
            pl.BlockSpec((Cemb, Cin), lambda i: (0, 0)),
            pl.BlockSpec((Cemb, 1), lambda i: (0, 0)),
            pl.BlockSpec((low_dim, Cin), lambda i: (0, 0)),
            pl.BlockSpec((low_dim, 1), lambda i: (0, 0)),
            pl.BlockSpec((n_cluster, low_dim), lambda i: (0, 0)),
            pl.BlockSpec((n_cluster, 1), lambda i: (0, 0)),
        ],
        out_specs=(
            pl.BlockSpec((bt, Cemb, HW), lambda i: (i, 0, 0)),
            pl.BlockSpec((1, low_dim, bt), lambda i: (i, 0, 0)),
            pl.BlockSpec((1, n_cluster, bt), lambda i: (i, 0, 0)),
        ),
        compiler_params=pltpu.CompilerParams(dimension_semantics=("parallel",)),
        cost_estimate=pl.CostEstimate(flops=flops, transcendentals=4 * B,
                                      bytes_accessed=bytes_accessed),
    )(x3, wbt_bf, bbt, wfet, bfet, wclt, bclt)

    metric = jnp.transpose(met_t, (0, 2, 1)).reshape(B, low_dim)
    cluster_n = jnp.transpose(clu_t, (0, 2, 1)).reshape(B, n_cluster)
    emb7x7 = emb3.reshape(B, Cemb, H, W)
    return metric, cluster_n, emb7x7


# speedup vs baseline: 1.2081x; 1.2081x over previous
import jax
import jax.numpy as jnp
from jax.experimental import pallas as pl
from jax.experimental.pallas import tpu as pltpu

HIGH = jax.lax.Precision.HIGHEST
HWP = 56


def _fused_kernel(bt, x_ref, wb_ref, bb_ref, wfe_ref, bfe_ref,
                  wcl_ref, bcl_ref, emb_ref, met_ref, clu_ref):
    x = x_ref[...]

    emb_ref[...] = jnp.dot(x.astype(jnp.bfloat16), wb_ref[...],
                           preferred_element_type=jnp.float32) + bb_ref[...]

    mt = x.shape[0]
    row_b = jax.lax.broadcasted_iota(jnp.int32, (bt, mt), 1) // HWP
    tgt_b = jax.lax.broadcasted_iota(jnp.int32, (bt, mt), 0)
    pool = jnp.where(row_b == tgt_b, jnp.float32(1.0), jnp.float32(0.0))
    x_mean = jnp.dot(pool, x, preferred_element_type=jnp.float32) * (1.0 / 49.0)

    feats = jnp.dot(x_mean, wfe_ref[...],
                    preferred_element_type=jnp.float32) + bfe_ref[...]
    inv_f = jax.lax.rsqrt(
        jnp.maximum(jnp.sum(feats * feats, axis=-1, keepdims=True), 1e-24))
    metric = feats * inv_f

    cluster = jnp.dot(metric, wcl_ref[...],
                      preferred_element_type=jnp.float32) + bcl_ref[...]
    inv_c = jax.lax.rsqrt(
        jnp.maximum(jnp.sum(cluster * cluster, axis=-1, keepdims=True), 1e-24))

    met_ref[...] = metric
    clu_ref[...] = cluster * inv_c


def kernel(x_nchw, w_base, b_base, w_feat, b_feat, bn_gamma, bn_beta,
           bn_rm, bn_rv, w_cl, b_cl):
    B, Cin, H, W = x_nchw.shape
    HW = H * W
    Cemb = w_base.shape[1]
    low_dim = w_feat.shape[1]
    n_cluster = w_cl.shape[1]

    bt = B
    for cand in (16, 8, 32, 40, 80):
        if B % cand == 0:
            bt = cand
            break
    n_tiles = B // bt
    Mt = bt * HWP

    x_hwc = jnp.transpose(x_nchw.reshape(B, Cin, HW), (0, 2, 1))
    x_pad = jnp.pad(x_hwc, ((0, 0), (0, HWP - HW), (0, 0)))
    x2d = x_pad.reshape(B * HWP, Cin)
    wb_bf = w_base.astype(jnp.bfloat16)

    w_feat_eff = jnp.dot(w_base, w_feat, precision=HIGH)
    b_feat_eff = jnp.dot(b_base, w_feat, precision=HIGH) + b_feat
    s = bn_gamma * jax.lax.rsqrt(bn_rv + 1e-5)
    w_cl_eff = w_cl * s.reshape(low_dim, 1)
    b_cl_eff = b_cl + jnp.dot(bn_beta - bn_rm * s, w_cl, precision=HIGH)

    flops = 2 * B * HWP * Cin * Cemb + 2 * B * Cin * low_dim \
        + 2 * B * low_dim * n_cluster
    bytes_accessed = 4 * (B * HWP * Cin + Cin * low_dim + low_dim
                          + low_dim * n_cluster + n_cluster
                          + B * HWP * Cemb + B * (low_dim + n_cluster)) \
        + 2 * Cin * Cemb

    body = lambda *refs: _fused_kernel(bt, *refs)
    emb2d, metric, cluster_n = pl.pallas_call(
        body,
        out_shape=(
            jax.ShapeDtypeStruct((B * HWP, Cemb), jnp.float32),
            jax.ShapeDtypeStruct((B, low_dim), jnp.float32),
            jax.ShapeDtypeStruct((B, n_cluster), jnp.float32),
        ),
        grid=(n_tiles,),
        in_specs=[
            pl.BlockSpec((Mt, Cin), lambda i: (i, 0)),
            pl.BlockSpec((Cin, Cemb), lambda i: (0, 0)),
            pl.BlockSpec((1, Cemb), lambda i: (0, 0)),
            pl.BlockSpec((Cin, low_dim), lambda i: (0, 0)),
            pl.BlockSpec((1, low_dim), lambda i: (0, 0)),
            pl.BlockSpec((low_dim, n_cluster), lambda i: (0, 0)),
            pl.BlockSpec((1, n_cluster), lambda i: (0, 0)),
        ],
        out_specs=(
            pl.BlockSpec((Mt, Cemb), lambda i: (i, 0)),
            pl.BlockSpec((bt, low_dim), lambda i: (i, 0)),
            pl.BlockSpec((bt, n_cluster), lambda i: (i, 0)),
        ),
        compiler_params=pltpu.CompilerParams(dimension_semantics=("parallel",)),
        cost_estimate=pl.CostEstimate(flops=flops, transcendentals=4 * B,
                                      bytes_accessed=bytes_accessed),
    )(x2d, wb_bf, b_base, w_feat_eff, b_feat_eff, w_cl_eff, b_cl_eff)

    emb7x7 = jnp.transpose(
        emb2d.reshape(B, HWP, Cemb)[:, :HW, :], (0, 2, 1)).reshape(B, Cemb, H, W)
    return metric, cluster_n, emb7x7


# speedup vs baseline: 1.4435x; 1.1949x over previous
import jax
import jax.numpy as jnp
from jax.experimental import pallas as pl
from jax.experimental.pallas import tpu as pltpu

HIGH = jax.lax.Precision.HIGHEST
SLOT = 64


def _fused_kernel(hw, bt, x_ref, wb_ref, bb_ref, wfe_ref, bfe_ref,
                  wcl_ref, bcl_ref, emb_ref, met_ref, clu_ref, xs_ref):
    for k in range(bt):
        xs_ref[pl.ds(k * SLOT, hw), :] = x_ref[k]

    acc = jnp.dot(xs_ref[...], wb_ref[...],
                  preferred_element_type=jnp.float32)
    bb = bb_ref[...]
    for k in range(bt):
        emb_ref[k] = (acc[k * SLOT:k * SLOT + hw, :] + bb).astype(jnp.bfloat16)

    x_mean = jnp.sum(x_ref[...].astype(jnp.float32), axis=1) * (1.0 / hw)

    feats = jnp.dot(x_mean, wfe_ref[...],
                    preferred_element_type=jnp.float32) + bfe_ref[...]
    inv_f = jax.lax.rsqrt(
        jnp.maximum(jnp.sum(feats * feats, axis=-1, keepdims=True), 1e-24))
    metric = feats * inv_f

    cluster = jnp.dot(metric, wcl_ref[...],
                      preferred_element_type=jnp.float32) + bcl_ref[...]
    inv_c = jax.lax.rsqrt(
        jnp.maximum(jnp.sum(cluster * cluster, axis=-1, keepdims=True), 1e-24))

    met_ref[...] = metric
    clu_ref[...] = cluster * inv_c


def kernel(x_nchw, w_base, b_base, w_feat, b_feat, bn_gamma, bn_beta,
           bn_rm, bn_rv, w_cl, b_cl):
    B, Cin, H, W = x_nchw.shape
    HW = H * W
    Cemb = w_base.shape[1]
    low_dim = w_feat.shape[1]
    n_cluster = w_cl.shape[1]

    bt = B
    for cand in (16, 8, 32, 40, 80):
        if B % cand == 0:
            bt = cand
            break
    n_tiles = B // bt

    x_hwc = jnp.transpose(
        x_nchw.reshape(B, Cin, HW), (0, 2, 1)).astype(jnp.bfloat16)
    wb_bf = w_base.astype(jnp.bfloat16)

    w_feat_eff = jnp.dot(w_base, w_feat, precision=HIGH)
    b_feat_eff = jnp.dot(b_base, w_feat, precision=HIGH) + b_feat
    s = bn_gamma * jax.lax.rsqrt(bn_rv + 1e-5)
    w_cl_eff = w_cl * s.reshape(low_dim, 1)
    b_cl_eff = b_cl + jnp.dot(bn_beta - bn_rm * s, w_cl, precision=HIGH)

    flops = 2 * B * SLOT * Cin * Cemb + 2 * B * Cin * low_dim \
        + 2 * B * low_dim * n_cluster
    bytes_accessed = 2 * (B * HW * Cin + Cin * Cemb + B * HW * Cemb) \
        + 4 * (Cin * low_dim + low_dim + low_dim * n_cluster + n_cluster
               + B * (low_dim + n_cluster))

    body = lambda *refs: _fused_kernel(HW, bt, *refs)
    emb3, metric, cluster_n = pl.pallas_call(
        body,
        out_shape=(
            jax.ShapeDtypeStruct((B, HW, Cemb), jnp.bfloat16),
            jax.ShapeDtypeStruct((B, low_dim), jnp.float32),
            jax.ShapeDtypeStruct((B, n_cluster), jnp.float32),
        ),
        grid=(n_tiles,),
        in_specs=[
            pl.BlockSpec((bt, HW, Cin), lambda i: (i, 0, 0)),
            pl.BlockSpec((Cin, Cemb), lambda i: (0, 0)),
            pl.BlockSpec((1, Cemb), lambda i: (0, 0)),
            pl.BlockSpec((Cin, low_dim), lambda i: (0, 0)),
            pl.BlockSpec((1, low_dim), lambda i: (0, 0)),
            pl.BlockSpec((low_dim, n_cluster), lambda i: (0, 0)),
            pl.BlockSpec((1, n_cluster), lambda i: (0, 0)),
        ],
        out_specs=(
            pl.BlockSpec((bt, HW, Cemb), lambda i: (i, 0, 0)),
            pl.BlockSpec((bt, low_dim), lambda i: (i, 0)),
            pl.BlockSpec((bt, n_cluster), lambda i: (i, 0)),
        ),
        scratch_shapes=[pltpu.VMEM((bt * SLOT, Cin), jnp.bfloat16)],
        compiler_params=pltpu.CompilerParams(dimension_semantics=("parallel",)),
        cost_estimate=pl.CostEstimate(flops=flops, transcendentals=4 * B,
                                      bytes_accessed=bytes_accessed),
    )(x_hwc, wb_bf, b_base, w_feat_eff, b_feat_eff, w_cl_eff, b_cl_eff)

    emb7x7 = jnp.transpose(
        emb3, (0, 2, 1)).astype(jnp.float32).reshape(B, Cemb, H, W)
    return metric, cluster_n, emb7x7


# speedup vs baseline: 1.7267x; 1.1962x over previous
import jax
import jax.numpy as jnp
from jax.experimental import pallas as pl
from jax.experimental.pallas import tpu as pltpu

HIGH = jax.lax.Precision.HIGHEST
SLOT = 64


def _fused_kernel(hw, bt, x_ref, wb_ref, bb_ref, wfe_ref, bfe_ref,
                  wcl_ref, bcl_ref, emb_ref, met_ref, clu_ref, xs_ref):
    for k in range(bt):
        xs_ref[pl.ds(k * SLOT, hw), :] = x_ref[k]

    acc = jnp.dot(xs_ref[...], wb_ref[...],
                  preferred_element_type=jnp.float32)
    bb = bb_ref[...]
    for k in range(bt):
        emb_ref[k] = acc[k * SLOT:k * SLOT + hw, :] + bb

    x_mean = jnp.sum(x_ref[...].astype(jnp.float32), axis=1) * (1.0 / hw)

    feats = jnp.dot(x_mean, wfe_ref[...],
                    preferred_element_type=jnp.float32) + bfe_ref[...]
    inv_f = jax.lax.rsqrt(
        jnp.maximum(jnp.sum(feats * feats, axis=-1, keepdims=True), 1e-24))
    metric = feats * inv_f

    cluster = jnp.dot(metric, wcl_ref[...],
                      preferred_element_type=jnp.float32) + bcl_ref[...]
    inv_c = jax.lax.rsqrt(
        jnp.maximum(jnp.sum(cluster * cluster, axis=-1, keepdims=True), 1e-24))

    met_ref[...] = metric
    clu_ref[...] = cluster * inv_c


def kernel(x_nchw, w_base, b_base, w_feat, b_feat, bn_gamma, bn_beta,
           bn_rm, bn_rv, w_cl, b_cl):
    B, Cin, H, W = x_nchw.shape
    HW = H * W
    Cemb = w_base.shape[1]
    low_dim = w_feat.shape[1]
    n_cluster = w_cl.shape[1]

    bt = B
    for cand in (16, 8, 32, 40, 80):
        if B % cand == 0:
            bt = cand
            break
    n_tiles = B // bt

    x_hwc = jnp.transpose(
        x_nchw.reshape(B, Cin, HW), (0, 2, 1)).astype(jnp.bfloat16)
    wb_bf = w_base.astype(jnp.bfloat16)

    w_feat_eff = jnp.dot(w_base, w_feat, precision=HIGH)
    b_feat_eff = jnp.dot(b_base, w_feat, precision=HIGH) + b_feat
    s = bn_gamma * jax.lax.rsqrt(bn_rv + 1e-5)
    w_cl_eff = w_cl * s.reshape(low_dim, 1)
    b_cl_eff = b_cl + jnp.dot(bn_beta - bn_rm * s, w_cl, precision=HIGH)

    flops = 2 * B * SLOT * Cin * Cemb + 2 * B * Cin * low_dim \
        + 2 * B * low_dim * n_cluster
    bytes_accessed = 2 * (B * HW * Cin + Cin * Cemb + B * HW * Cemb) \
        + 4 * (Cin * low_dim + low_dim + low_dim * n_cluster + n_cluster
               + B * (low_dim + n_cluster))

    body = lambda *refs: _fused_kernel(HW, bt, *refs)
    emb3, metric, cluster_n = pl.pallas_call(
        body,
        out_shape=(
            jax.ShapeDtypeStruct((B, HW, Cemb), jnp.float32),
            jax.ShapeDtypeStruct((B, low_dim), jnp.float32),
            jax.ShapeDtypeStruct((B, n_cluster), jnp.float32),
        ),
        grid=(n_tiles,),
        in_specs=[
            pl.BlockSpec((bt, HW, Cin), lambda i: (i, 0, 0)),
            pl.BlockSpec((Cin, Cemb), lambda i: (0, 0)),
            pl.BlockSpec((1, Cemb), lambda i: (0, 0)),
            pl.BlockSpec((Cin, low_dim), lambda i: (0, 0)),
            pl.BlockSpec((1, low_dim), lambda i: (0, 0)),
            pl.BlockSpec((low_dim, n_cluster), lambda i: (0, 0)),
            pl.BlockSpec((1, n_cluster), lambda i: (0, 0)),
        ],
        out_specs=(
            pl.BlockSpec((bt, HW, Cemb), lambda i: (i, 0, 0)),
            pl.BlockSpec((bt, low_dim), lambda i: (i, 0)),
            pl.BlockSpec((bt, n_cluster), lambda i: (i, 0)),
        ),
        scratch_shapes=[pltpu.VMEM((bt * SLOT, Cin), jnp.bfloat16)],
        compiler_params=pltpu.CompilerParams(dimension_semantics=("parallel",)),
        cost_estimate=pl.CostEstimate(flops=flops, transcendentals=4 * B,
                                      bytes_accessed=bytes_accessed),
    )(x_hwc, wb_bf, b_base, w_feat_eff, b_feat_eff, w_cl_eff, b_cl_eff)

    emb7x7 = jnp.transpose(emb3, (0, 2, 1)).reshape(B, Cemb, H, W)
    return metric, cluster_n, emb7x7


# speedup vs baseline: 3.6233x; 2.0984x over previous
import jax
import jax.numpy as jnp
from jax.experimental import pallas as pl
from jax.experimental.pallas import tpu as pltpu

HIGH = jax.lax.Precision.HIGHEST


def _fused_kernel(hw, bt, x_ref, wb_ref, bb_ref, wfe_ref, bfe_ref,
                  wcl_ref, bcl_ref, emb_ref, met_ref, clu_ref):
    cin = x_ref.shape[2]
    cemb = wb_ref.shape[1]
    x2 = x_ref[...].reshape(hw * bt, cin)

    acc = jnp.dot(x2, wb_ref[...],
                  preferred_element_type=jnp.float32) + bb_ref[...]
    emb_ref[...] = acc.reshape(hw, bt, cemb)

    x_mean = jnp.sum(x_ref[...].astype(jnp.float32), axis=0) * (1.0 / hw)

    feats = jnp.dot(x_mean, wfe_ref[...],
                    preferred_element_type=jnp.float32) + bfe_ref[...]
    inv_f = jax.lax.rsqrt(
        jnp.maximum(jnp.sum(feats * feats, axis=-1, keepdims=True), 1e-24))
    metric = feats * inv_f

    cluster = jnp.dot(metric, wcl_ref[...],
                      preferred_element_type=jnp.float32) + bcl_ref[...]
    inv_c = jax.lax.rsqrt(
        jnp.maximum(jnp.sum(cluster * cluster, axis=-1, keepdims=True), 1e-24))

    met_ref[...] = metric
    clu_ref[...] = cluster * inv_c


def kernel(x_nchw, w_base, b_base, w_feat, b_feat, bn_gamma, bn_beta,
           bn_rm, bn_rv, w_cl, b_cl):
    B, Cin, H, W = x_nchw.shape
    HW = H * W
    Cemb = w_base.shape[1]
    low_dim = w_feat.shape[1]
    n_cluster = w_cl.shape[1]

    bt = B
    for cand in (16, 8, 32, 40, 80):
        if B % cand == 0:
            bt = cand
            break
    n_tiles = B // bt

    x_hbc = jnp.transpose(
        x_nchw.reshape(B, Cin, HW), (2, 0, 1)).astype(jnp.bfloat16)
    wb_bf = w_base.astype(jnp.bfloat16)

    w_feat_eff = jnp.dot(w_base, w_feat, precision=HIGH)
    b_feat_eff = jnp.dot(b_base, w_feat, precision=HIGH) + b_feat
    s = bn_gamma * jax.lax.rsqrt(bn_rv + 1e-5)
    w_cl_eff = w_cl * s.reshape(low_dim, 1)
    b_cl_eff = b_cl + jnp.dot(bn_beta - bn_rm * s, w_cl, precision=HIGH)

    flops = 2 * B * HW * Cin * Cemb + 2 * B * Cin * low_dim \
        + 2 * B * low_dim * n_cluster
    bytes_accessed = 2 * (B * HW * Cin + Cin * Cemb) \
        + 4 * (B * HW * Cemb + Cin * low_dim + low_dim
               + low_dim * n_cluster + n_cluster + B * (low_dim + n_cluster))

    body = lambda *refs: _fused_kernel(HW, bt, *refs)
    emb_hbc, metric, cluster_n = pl.pallas_call(
        body,
        out_shape=(
            jax.ShapeDtypeStruct((HW, B, Cemb), jnp.float32),
            jax.ShapeDtypeStruct((B, low_dim), jnp.float32),
            jax.ShapeDtypeStruct((B, n_cluster), jnp.float32),
        ),
        grid=(n_tiles,),
        in_specs=[
            pl.BlockSpec((HW, bt, Cin), lambda i: (0, i, 0)),
            pl.BlockSpec((Cin, Cemb), lambda i: (0, 0)),
            pl.BlockSpec((1, Cemb), lambda i: (0, 0)),
            pl.BlockSpec((Cin, low_dim), lambda i: (0, 0)),
            pl.BlockSpec((1, low_dim), lambda i: (0, 0)),
            pl.BlockSpec((low_dim, n_cluster), lambda i: (0, 0)),
            pl.BlockSpec((1, n_cluster), lambda i: (0, 0)),
        ],
        out_specs=(
            pl.BlockSpec((HW, bt, Cemb), lambda i: (0, i, 0)),
            pl.BlockSpec((bt, low_dim), lambda i: (i, 0)),
            pl.BlockSpec((bt, n_cluster), lambda i: (i, 0)),
        ),
        compiler_params=pltpu.CompilerParams(dimension_semantics=("parallel",)),
        cost_estimate=pl.CostEstimate(flops=flops, transcendentals=4 * B,
                                      bytes_accessed=bytes_accessed),
    )(x_hbc, wb_bf, b_base, w_feat_eff, b_feat_eff, w_cl_eff, b_cl_eff)

    emb7x7 = jnp.transpose(emb_hbc, (1, 2, 0)).reshape(B, Cemb, H, W)
    return metric, cluster_n, emb7x7


# speedup vs baseline: 4.4001x; 1.2144x over previous
import jax
import jax.numpy as jnp
from jax.experimental import pallas as pl
from jax.experimental.pallas import tpu as pltpu

HIGH = jax.lax.Precision.HIGHEST


def _fused_kernel(hw, bt, x_ref, wb_ref, bb_ref, wfe_ref, bfe_ref,
                  wcl_ref, bcl_ref, emb_ref, met_ref, clu_ref):
    cin = x_ref.shape[2]
    cemb = wb_ref.shape[1]
    x = x_ref[...]
    x2 = x.reshape(hw * bt, cin)

    acc = jnp.dot(x2.astype(jnp.bfloat16), wb_ref[...],
                  preferred_element_type=jnp.float32) + bb_ref[...]
    emb_ref[...] = acc.reshape(hw, bt, cemb)

    x_mean = jnp.sum(x, axis=0) * (1.0 / hw)

    feats = jnp.dot(x_mean, wfe_ref[...],
                    preferred_element_type=jnp.float32) + bfe_ref[...]
    inv_f = jax.lax.rsqrt(
        jnp.maximum(jnp.sum(feats * feats, axis=-1, keepdims=True), 1e-24))
    metric = feats * inv_f

    cluster = jnp.dot(metric, wcl_ref[...],
                      preferred_element_type=jnp.float32) + bcl_ref[...]
    inv_c = jax.lax.rsqrt(
        jnp.maximum(jnp.sum(cluster * cluster, axis=-1, keepdims=True), 1e-24))

    met_ref[...] = metric
    clu_ref[...] = cluster * inv_c


def kernel(x_nchw, w_base, b_base, w_feat, b_feat, bn_gamma, bn_beta,
           bn_rm, bn_rv, w_cl, b_cl):
    B, Cin, H, W = x_nchw.shape
    HW = H * W
    Cemb = w_base.shape[1]
    low_dim = w_feat.shape[1]
    n_cluster = w_cl.shape[1]

    bt = B
    for cand in (16, 8, 32, 40, 80):
        if B % cand == 0:
            bt = cand
            break
    n_tiles = B // bt

    x_hbc = jnp.transpose(x_nchw.reshape(B, Cin, HW), (2, 0, 1))
    wb_bf = w_base.astype(jnp.bfloat16)

    w_feat_eff = jnp.dot(w_base, w_feat, precision=HIGH)
    b_feat_eff = jnp.dot(b_base, w_feat, precision=HIGH) + b_feat
    s = bn_gamma * jax.lax.rsqrt(bn_rv + 1e-5)
    w_cl_eff = w_cl * s.reshape(low_dim, 1)
    b_cl_eff = b_cl + jnp.dot(bn_beta - bn_rm * s, w_cl, precision=HIGH)

    flops = 2 * B * HW * Cin * Cemb + 2 * B * Cin * low_dim \
        + 2 * B * low_dim * n_cluster
    bytes_accessed = 2 * Cin * Cemb \
        + 4 * (B * HW * Cin + B * HW * Cemb + Cin * low_dim + low_dim
               + low_dim * n_cluster + n_cluster + B * (low_dim + n_cluster))

    body = lambda *refs: _fused_kernel(HW, bt, *refs)
    emb_hbc, metric, cluster_n = pl.pallas_call(
        body,
        out_shape=(
            jax.ShapeDtypeStruct((HW, B, Cemb), jnp.float32),
            jax.ShapeDtypeStruct((B, low_dim), jnp.float32),
            jax.ShapeDtypeStruct((B, n_cluster), jnp.float32),
        ),
        grid=(n_tiles,),
        in_specs=[
            pl.BlockSpec((HW, bt, Cin), lambda i: (0, i, 0)),
            pl.BlockSpec((Cin, Cemb), lambda i: (0, 0)),
            pl.BlockSpec((1, Cemb), lambda i: (0, 0)),
            pl.BlockSpec((Cin, low_dim), lambda i: (0, 0)),
            pl.BlockSpec((1, low_dim), lambda i: (0, 0)),
            pl.BlockSpec((low_dim, n_cluster), lambda i: (0, 0)),
            pl.BlockSpec((1, n_cluster), lambda i: (0, 0)),
        ],
        out_specs=(
            pl.BlockSpec((HW, bt, Cemb), lambda i: (0, i, 0)),
            pl.BlockSpec((bt, low_dim), lambda i: (i, 0)),
            pl.BlockSpec((bt, n_cluster), lambda i: (i, 0)),
        ),
        compiler_params=pltpu.CompilerParams(dimension_semantics=("parallel",)),
        cost_estimate=pl.CostEstimate(flops=flops, transcendentals=4 * B,
                                      bytes_accessed=bytes_accessed),
    )(x_hbc, wb_bf, b_base, w_feat_eff, b_feat_eff, w_cl_eff, b_cl_eff)

    emb7x7 = jnp.transpose(emb_hbc, (1, 2, 0)).reshape(B, Cemb, H, W)
    return metric, cluster_n, emb7x7


# speedup vs baseline: 6.0959x; 1.3854x over previous
import jax
import jax.numpy as jnp
from jax.experimental import pallas as pl
from jax.experimental.pallas import tpu as pltpu


def _fused_kernel(hw, bt, x_ref, wb_ref, bb_ref, wf_ref, bf_ref, g_ref,
                  be_ref, rm_ref, rv_ref, wc_ref, bc_ref,
                  emb_ref, met_ref, clu_ref):
    cin = x_ref.shape[2]
    cemb = wb_ref.shape[1]
    x = x_ref[...]
    x2 = x.reshape(hw * bt, cin)
    wb_bf = wb_ref[...].astype(jnp.bfloat16)

    acc = jnp.dot(x2.astype(jnp.bfloat16), wb_bf,
                  preferred_element_type=jnp.float32) + bb_ref[...]
    emb_ref[...] = acc.reshape(hw, bt, cemb)

    x_mean = jnp.sum(x, axis=0) * (1.0 / hw)
    emb_mean = jnp.dot(x_mean.astype(jnp.bfloat16), wb_bf,
                       preferred_element_type=jnp.float32) + bb_ref[...]
    feats = jnp.dot(emb_mean.astype(jnp.bfloat16),
                    wf_ref[...].astype(jnp.bfloat16),
                    preferred_element_type=jnp.float32) + bf_ref[...]
    inv_f = jax.lax.rsqrt(
        jnp.maximum(jnp.sum(feats * feats, axis=-1, keepdims=True), 1e-24))
    metric = feats * inv_f

    s = g_ref[...] * jax.lax.rsqrt(rv_ref[...] + 1e-5)
    bn = metric * s + (be_ref[...] - rm_ref[...] * s)
    cluster = jnp.dot(bn.astype(jnp.bfloat16),
                      wc_ref[...].astype(jnp.bfloat16),
                      preferred_element_type=jnp.float32) + bc_ref[...]
    inv_c = jax.lax.rsqrt(
        jnp.maximum(jnp.sum(cluster * cluster, axis=-1, keepdims=True), 1e-24))

    met_ref[...] = metric
    clu_ref[...] = cluster * inv_c


def kernel(x_nchw, w_base, b_base, w_feat, b_feat, bn_gamma, bn_beta,
           bn_rm, bn_rv, w_cl, b_cl):
    B, Cin, H, W = x_nchw.shape
    HW = H * W
    Cemb = w_base.shape[1]
    low_dim = w_feat.shape[1]
    n_cluster = w_cl.shape[1]

    bt = B
    for cand in (16, 8, 32, 40, 80):
        if B % cand == 0:
            bt = cand
            break
    n_tiles = B // bt

    x_hbc = jnp.transpose(x_nchw.reshape(B, Cin, HW), (2, 0, 1))

    flops = 2 * B * HW * Cin * Cemb + 2 * B * Cin * Cemb \
        + 2 * B * Cemb * low_dim + 2 * B * low_dim * n_cluster
    bytes_accessed = 4 * (B * HW * Cin + B * HW * Cemb + Cin * Cemb
                          + Cemb * low_dim + low_dim * n_cluster
                          + B * (low_dim + n_cluster))

    body = lambda *refs: _fused_kernel(HW, bt, *refs)
    emb_hbc, metric, cluster_n = pl.pallas_call(
        body,
        out_shape=(
            jax.ShapeDtypeStruct((HW, B, Cemb), jnp.float32),
            jax.ShapeDtypeStruct((B, low_dim), jnp.float32),
            jax.ShapeDtypeStruct((B, n_cluster), jnp.float32),
        ),
        grid=(n_tiles,),
        in_specs=[
            pl.BlockSpec((HW, bt, Cin), lambda i: (0, i, 0)),
            pl.BlockSpec((Cin, Cemb), lambda i: (0, 0)),
            pl.BlockSpec((1, Cemb), lambda i: (0, 0)),
            pl.BlockSpec((Cemb, low_dim), lambda i: (0, 0)),
            pl.BlockSpec((1, low_dim), lambda i: (0, 0)),
            pl.BlockSpec((1, low_dim), lambda i: (0, 0)),
            pl.BlockSpec((1, low_dim), lambda i: (0, 0)),
            pl.BlockSpec((1, low_dim), lambda i: (0, 0)),
            pl.BlockSpec((1, low_dim), lambda i: (0, 0)),
            pl.BlockSpec((low_dim, n_cluster), lambda i: (0, 0)),
            pl.BlockSpec((1, n_cluster), lambda i: (0, 0)),
        ],
        out_specs=(
            pl.BlockSpec((HW, bt, Cemb), lambda i: (0, i, 0)),
            pl.BlockSpec((bt, low_dim), lambda i: (i, 0)),
            pl.BlockSpec((bt, n_cluster), lambda i: (i, 0)),
        ),
        compiler_params=pltpu.CompilerParams(dimension_semantics=("parallel",)),
        cost_estimate=pl.CostEstimate(flops=flops, transcendentals=4 * B,
                                      bytes_accessed=bytes_accessed),
    )(x_hbc, w_base, b_base, w_feat, b_feat, bn_gamma, bn_beta,
      bn_rm, bn_rv, w_cl, b_cl)

    emb7x7 = jnp.transpose(emb_hbc, (1, 2, 0)).reshape(B, Cemb, H, W)
    return metric, cluster_n, emb7x7
